# Initial kernel scaffold; baseline (speedup 1.0000x reference)
import jax
import jax.numpy as jnp
from jax.experimental import pallas as pl


def kernel(val_table, coord_table, pos_table, input_ids, coord_ids, pos_ids):
    raise NotImplementedError("write your pallas kernel here")


# speedup vs baseline: 1.1909x; 1.1909x over previous
import functools

import jax
import jax.numpy as jnp
from jax import lax
from jax.experimental import pallas as pl
from jax.experimental.pallas import tpu as pltpu
from jax.experimental.pallas import tpu_sc as plsc

_B, _L, _D = 1024, 50, 1536
_N = _B * _L
_NC, _NS = 2, 16
_NW = _NC * _NS
_PER_W = _N // _NW
_C = 16
_CHUNKS = _PER_W // _C
_VPR = _D // 16


def _sc_body(val_hbm, coord_hbm, pos_hbm, iv_hbm, ic_hbm, ip_hbm, out_hbm,
             iv_v, ic_v, ip_v, a_v, b_v, c_v, sem):
    wid = lax.axis_index("s") * _NC + lax.axis_index("c")
    base = wid * _PER_W

    pltpu.sync_copy(iv_hbm.at[pl.ds(base, _PER_W)], iv_v)
    pltpu.sync_copy(ic_hbm.at[pl.ds(base, _PER_W)], ic_v)
    pltpu.sync_copy(ip_hbm.at[pl.ds(base, _PER_W)], ip_v)

    def chunk_body(i, carry):
        t0 = i * _C
        cp1 = pltpu.async_copy(val_hbm.at[iv_v.at[pl.ds(t0, _C)]], a_v, sem)
        cp2 = pltpu.async_copy(coord_hbm.at[ic_v.at[pl.ds(t0, _C)]], b_v, sem)
        cp3 = pltpu.async_copy(pos_hbm.at[ip_v.at[pl.ds(t0, _C)]], c_v, sem)
        cp1.wait()
        cp2.wait()
        cp3.wait()

        def add_body(k, acc):
            t = k // _VPR
            j = (k % _VPR) * 16
            a_v[t, pl.ds(j, 16)] = (a_v[t, pl.ds(j, 16)]
                                    + b_v[t, pl.ds(j, 16)]
                                    + c_v[t, pl.ds(j, 16)])
            return acc

        lax.fori_loop(0, _C * _VPR, add_body, 0)
        pltpu.sync_copy(a_v, out_hbm.at[pl.ds(base + t0, _C)])
        return carry

    lax.fori_loop(0, _CHUNKS, chunk_body, 0)


@functools.partial(jax.jit, static_argnums=())
def _run(val_table, coord_table, pos_table, iv, ic, ip):
    mesh = plsc.VectorSubcoreMesh(core_axis_name="c", subcore_axis_name="s")
    f = functools.partial(
        pl.kernel,
        out_type=jax.ShapeDtypeStruct((_N, _D), jnp.float32),
        mesh=mesh,
        scratch_types=[
            pltpu.VMEM((_PER_W,), jnp.int32),
            pltpu.VMEM((_PER_W,), jnp.int32),
            pltpu.VMEM((_PER_W,), jnp.int32),
            pltpu.VMEM((_C, _D), jnp.float32),
            pltpu.VMEM((_C, _D), jnp.float32),
            pltpu.VMEM((_C, _D), jnp.float32),
            pltpu.SemaphoreType.DMA,
        ],
    )(_sc_body)
    return f(val_table, coord_table, pos_table, iv, ic, ip)


def kernel(val_table, coord_table, pos_table, input_ids, coord_ids, pos_ids):
    iv = input_ids.reshape(-1).astype(jnp.int32)
    ic = coord_ids.reshape(-1).astype(jnp.int32)
    ip = pos_ids.reshape(-1).astype(jnp.int32)
    out = _run(val_table, coord_table, pos_table, iv, ic, ip)
    return out.reshape(_B, _L, _D)


# speedup vs baseline: 1.4626x; 1.2281x over previous
import functools

import jax
import jax.numpy as jnp
from jax import lax
from jax.experimental import pallas as pl
from jax.experimental.pallas import tpu as pltpu
from jax.experimental.pallas import tpu_sc as plsc

_B, _L, _D = 1024, 50, 1536
_N = _B * _L
_NC, _NS = 2, 16
_NW = _NC * _NS
_PER_W = _N // _NW
_C = 8
_CHUNKS = _PER_W // _C
_NBUF = 2
_VPR = _D // 16
_NVAL = 70
_NCOORD = 130


def _sc_body(tbl_hbm, idx3_hbm, out_hbm,
             idx_v, rows0, rows1, outb0, outb1, gs0, gs1, os0, os1):
    rows = (rows0, rows1)
    outs = (outb0, outb1)
    gsems = (gs0, gs1)
    osems = (os0, os1)
    wid = lax.axis_index("s") * _NC + lax.axis_index("c")
    base = wid * _PER_W

    pltpu.sync_copy(idx3_hbm.at[pl.ds(base * 3, _PER_W * 3)], idx_v)

    for b in range(_NBUF):
        pltpu.async_copy(
            tbl_hbm.at[idx_v.at[pl.ds(b * 3 * _C, 3 * _C)]], rows[b], gsems[b])

    def outer(h, carry):
        for b in range(_NBUF):
            i = h * _NBUF + b
            pltpu.make_async_copy(
                tbl_hbm.at[idx_v.at[pl.ds(i * 3 * _C, 3 * _C)]],
                rows[b], gsems[b]).wait()

            @pl.when(i >= _NBUF)
            def _():
                pltpu.make_async_copy(
                    outs[b], out_hbm.at[pl.ds(base, _C)], osems[b]).wait()

            def add_t(t, c2):
                r0 = 3 * t
                for j in range(_VPR):
                    sl = pl.ds(j * 16, 16)
                    outs[b][t, sl] = (rows[b][r0, sl]
                                      + rows[b][r0 + 1, sl]
                                      + rows[b][r0 + 2, sl])
                return c2

            lax.fori_loop(0, _C, add_t, 0)

            pltpu.async_copy(
                outs[b], out_hbm.at[pl.ds(base + i * _C, _C)], osems[b])

            @pl.when(i + _NBUF < _CHUNKS)
            def _():
                pltpu.async_copy(
                    tbl_hbm.at[idx_v.at[pl.ds((i + _NBUF) * 3 * _C, 3 * _C)]],
                    rows[b], gsems[b])
        return carry

    lax.fori_loop(0, _CHUNKS // _NBUF, outer, 0)

    for b in range(_NBUF):
        pltpu.make_async_copy(
            outs[b], out_hbm.at[pl.ds(base, _C)], osems[b]).wait()


@jax.jit
def _run(tbl, idx3):
    mesh = plsc.VectorSubcoreMesh(core_axis_name="c", subcore_axis_name="s")
    f = functools.partial(
        pl.kernel,
        out_type=jax.ShapeDtypeStruct((_N, _D), jnp.float32),
        mesh=mesh,
        scratch_types=[
            pltpu.VMEM((_PER_W * 3,), jnp.int32),
            pltpu.VMEM((3 * _C, _D), jnp.float32),
            pltpu.VMEM((3 * _C, _D), jnp.float32),
            pltpu.VMEM((_C, _D), jnp.float32),
            pltpu.VMEM((_C, _D), jnp.float32),
            pltpu.SemaphoreType.DMA,
            pltpu.SemaphoreType.DMA,
            pltpu.SemaphoreType.DMA,
            pltpu.SemaphoreType.DMA,
        ],
    )(_sc_body)
    return f(tbl, idx3)


def kernel(val_table, coord_table, pos_table, input_ids, coord_ids, pos_ids):
    tbl = jnp.concatenate([val_table, coord_table, pos_table], axis=0)
    iv = input_ids.reshape(-1).astype(jnp.int32)
    ic = coord_ids.reshape(-1).astype(jnp.int32) + _NVAL
    ip = pos_ids.reshape(-1).astype(jnp.int32) + (_NVAL + _NCOORD)
    idx3 = jnp.stack([iv, ic, ip], axis=1).reshape(-1)
    out = _run(tbl, idx3)
    return out.reshape(_B, _L, _D)


# speedup vs baseline: 1.4995x; 1.0252x over previous
import functools

import jax
import jax.numpy as jnp
from jax import lax
from jax.experimental import pallas as pl
from jax.experimental.pallas import tpu as pltpu
from jax.experimental.pallas import tpu_sc as plsc

_B, _L, _D = 1024, 50, 1536
_N = _B * _L
_NC, _NS = 2, 16
_NW = _NC * _NS
_PER_W = _N // _NW
_C = 8
_CHUNKS = _PER_W // _C
_NBUF = 2
_VPR = _D // 16
_NVAL = 70
_NCOORD = 130


def _sc_body(tbl_hbm, idx3_hbm, out_hbm,
             idx_v, rows0, rows1, outb0, outb1, gs0, gs1, os0, os1):
    rows = (rows0, rows1)
    outs = (outb0, outb1)
    gsems = (gs0, gs1)
    osems = (os0, os1)
    wid = lax.axis_index("s") * _NC + lax.axis_index("c")
    base = wid * _PER_W

    pltpu.sync_copy(idx3_hbm.at[pl.ds(base * 3, _PER_W * 3)], idx_v)

    for b in range(_NBUF):
        pltpu.async_copy(
            tbl_hbm.at[idx_v.at[pl.ds(b * 3 * _C, 3 * _C)]], rows[b], gsems[b])

    def outer(h, carry):
        for b in range(_NBUF):
            i = h * _NBUF + b
            pltpu.make_async_copy(
                tbl_hbm.at[idx_v.at[pl.ds(i * 3 * _C, 3 * _C)]],
                rows[b], gsems[b]).wait()

            @pl.when(i >= _NBUF)
            def _():
                pltpu.make_async_copy(
                    outs[b], out_hbm.at[pl.ds(base, _C)], osems[b]).wait()

            @plsc.parallel_loop(0, _VPR, unroll=2)
            def add_j(j):
                sl = pl.ds(j * 16, 16)
                for t in range(_C):
                    outs[b][t, sl] = (rows[b][3 * t, sl]
                                      + rows[b][3 * t + 1, sl]
                                      + rows[b][3 * t + 2, sl])

            pltpu.async_copy(
                outs[b], out_hbm.at[pl.ds(base + i * _C, _C)], osems[b])

            @pl.when(i + _NBUF < _CHUNKS)
            def _():
                pltpu.async_copy(
                    tbl_hbm.at[idx_v.at[pl.ds((i + _NBUF) * 3 * _C, 3 * _C)]],
                    rows[b], gsems[b])
        return carry

    lax.fori_loop(0, _CHUNKS // _NBUF, outer, 0)

    for b in range(_NBUF):
        pltpu.make_async_copy(
            outs[b], out_hbm.at[pl.ds(base, _C)], osems[b]).wait()


@jax.jit
def _run(tbl, idx3):
    mesh = plsc.VectorSubcoreMesh(core_axis_name="c", subcore_axis_name="s")
    f = functools.partial(
        pl.kernel,
        out_type=jax.ShapeDtypeStruct((_N, _D), jnp.float32),
        mesh=mesh,
        scratch_types=[
            pltpu.VMEM((_PER_W * 3,), jnp.int32),
            pltpu.VMEM((3 * _C, _D), jnp.float32),
            pltpu.VMEM((3 * _C, _D), jnp.float32),
            pltpu.VMEM((_C, _D), jnp.float32),
            pltpu.VMEM((_C, _D), jnp.float32),
            pltpu.SemaphoreType.DMA,
            pltpu.SemaphoreType.DMA,
            pltpu.SemaphoreType.DMA,
            pltpu.SemaphoreType.DMA,
        ],
    )(_sc_body)
    return f(tbl, idx3)


def kernel(val_table, coord_table, pos_table, input_ids, coord_ids, pos_ids):
    tbl = jnp.concatenate([val_table, coord_table, pos_table], axis=0)
    iv = input_ids.reshape(-1).astype(jnp.int32)
    ic = coord_ids.reshape(-1).astype(jnp.int32) + _NVAL
    ip = pos_ids.reshape(-1).astype(jnp.int32) + (_NVAL + _NCOORD)
    idx3 = jnp.stack([iv, ic, ip], axis=1).reshape(-1)
    out = _run(tbl, idx3)
    return out.reshape(_B, _L, _D)


# speedup vs baseline: 2.5275x; 1.6856x over previous
import functools

import jax
import jax.numpy as jnp
from jax import lax
from jax.experimental import pallas as pl
from jax.experimental.pallas import tpu as pltpu
from jax.experimental.pallas import tpu_sc as plsc

_B, _L, _D = 1024, 50, 1536
_N = _B * _L
_NC, _NS = 2, 16
_NW = _NC * _NS
_NDS = 4
_DS = _D // _NDS
_NTG = _NW // _NDS
_PER_T = _N // _NTG
_C = 16
_CHUNKS = _PER_T // _C
_JPT = _DS // 16
_V = 224
_NVAL = 70
_NCOORD = 130


def _sc_body(tbl_hbm, idx3_hbm, out_hbm, tbl_v, idx_v, outb0, outb1, os0, os1):
    outs = (outb0, outb1)
    osems = (os0, os1)
    wid = lax.axis_index("s") * _NC + lax.axis_index("c")
    dgrp = wid % _NDS
    tgrp = wid // _NDS
    d0 = dgrp * _DS
    tbase = tgrp * _PER_T

    pltpu.sync_copy(tbl_hbm.at[:, pl.ds(d0, _DS)], tbl_v)
    pltpu.sync_copy(idx3_hbm.at[pl.ds(tbase * 3, _PER_T * 3)],
                    idx_v.at[pl.ds(0, _PER_T * 3)])

    def chunk(h, carry):
      for b in range(2):
        i = h * 2 + b
        t0 = i * _C

        @pl.when(i >= 2)
        def _():
            pltpu.make_async_copy(
                outs[b],
                out_hbm.at[pl.ds(tbase, _C), pl.ds(d0, _DS)],
                osems[b]).wait()

        @plsc.parallel_loop(0, _C, unroll=1)
        def per_token(t):
            tok3 = (t0 + t) * 3
            v3 = idx_v[pl.ds(tok3, 16)]
            rv = v3[0]
            rc = v3[1]
            rp = v3[2]
            for j in range(_JPT):
                sl = pl.ds(j * 16, 16)
                outs[b][t, sl] = (tbl_v[rv, sl]
                                  + tbl_v[rc, sl]
                                  + tbl_v[rp, sl])

        pltpu.async_copy(
            outs[b],
            out_hbm.at[pl.ds(tbase + t0, _C), pl.ds(d0, _DS)],
            osems[b])
      return carry

    lax.fori_loop(0, _CHUNKS // 2, chunk, 0)

    for b in range(2):
        pltpu.make_async_copy(
            outs[b],
            out_hbm.at[pl.ds(tbase, _C), pl.ds(d0, _DS)],
            osems[b]).wait()


@jax.jit
def _run(tbl, idx3):
    mesh = plsc.VectorSubcoreMesh(core_axis_name="c", subcore_axis_name="s")
    f = functools.partial(
        pl.kernel,
        out_type=jax.ShapeDtypeStruct((_N, _D), jnp.float32),
        mesh=mesh,
        scratch_types=[
            pltpu.VMEM((_V, _DS), jnp.float32),
            pltpu.VMEM((_PER_T * 3 + 16,), jnp.int32),
            pltpu.VMEM((_C, _DS), jnp.float32),
            pltpu.VMEM((_C, _DS), jnp.float32),
            pltpu.SemaphoreType.DMA,
            pltpu.SemaphoreType.DMA,
        ],
    )(_sc_body)
    return f(tbl, idx3)


def kernel(val_table, coord_table, pos_table, input_ids, coord_ids, pos_ids):
    tbl = jnp.concatenate(
        [val_table, coord_table, pos_table,
         jnp.zeros((_V - 219, _D), jnp.float32)], axis=0)
    iv = input_ids.reshape(-1).astype(jnp.int32)
    ic = coord_ids.reshape(-1).astype(jnp.int32) + _NVAL
    ip = pos_ids.reshape(-1).astype(jnp.int32) + (_NVAL + _NCOORD)
    idx3 = jnp.stack([iv, ic, ip], axis=1).reshape(-1)
    out = _run(tbl, idx3)
    return out.reshape(_B, _L, _D)


# speedup vs baseline: 3.6582x; 1.4474x over previous
import functools

import jax
import jax.numpy as jnp
from jax import lax
from jax.experimental import pallas as pl
from jax.experimental.pallas import tpu as pltpu
from jax.experimental.pallas import tpu_sc as plsc

_B, _L, _D = 1024, 50, 1536
_N = _B * _L
_NC, _NS = 2, 16
_NW = _NC * _NS
_NDS = 4
_DS = _D // _NDS
_NTG = _NW // _NDS
_ROWS_T = _B // _NTG
_IGRP = 8
_JPT = _DS // 16
_V = 224
_NVAL = 70
_NCOORD = 130


def _sc_body(tbl_hbm, idx3_hbm, out_hbm, tbl_v, idx_v, outb0, outb1, os0, os1):
    outs = (outb0, outb1)
    osems = (os0, os1)
    wid = lax.axis_index("s") * _NC + lax.axis_index("c")
    dgrp = wid % _NDS
    tgrp = wid // _NDS
    d0 = dgrp * _DS
    rbase = tgrp * _ROWS_T

    pltpu.sync_copy(tbl_hbm.at[:, pl.ds(d0, _DS)], tbl_v)

    def chunk(hp, carry):
      for bb in range(2):
        h = hp * 2 + bb

        @pl.when(lax.rem(h, _IGRP) == 0)
        def _():
            g = lax.div(h, _IGRP)
            off = tgrp * (_ROWS_T * _L * 3) + g * (_IGRP * _L * 3)
            pltpu.sync_copy(
                idx3_hbm.at[pl.ds(off, _IGRP * _L * 3)],
                idx_v.at[pl.ds(0, _IGRP * _L * 3)])

        @pl.when(h >= 2)
        def _():
            pltpu.make_async_copy(
                outs[bb],
                out_hbm.at[rbase, :, pl.ds(d0, _DS)],
                osems[bb]).wait()

        hh = lax.rem(h, _IGRP)

        @plsc.parallel_loop(0, _L, unroll=1)
        def per_token(t):
            tok3 = (hh * _L + t) * 3
            v3 = idx_v[pl.ds(tok3, 16)]
            rv = v3[0]
            rc = v3[1]
            rp = v3[2]
            for j in range(_JPT):
                sl = pl.ds(j * 16, 16)
                outs[bb][t, sl] = (tbl_v[rv, sl]
                                   + tbl_v[rc, sl]
                                   + tbl_v[rp, sl])

        pltpu.async_copy(
            outs[bb],
            out_hbm.at[rbase + h, :, pl.ds(d0, _DS)],
            osems[bb])
      return carry

    lax.fori_loop(0, _ROWS_T // 2, chunk, 0)

    for bb in range(2):
        pltpu.make_async_copy(
            outs[bb],
            out_hbm.at[rbase, :, pl.ds(d0, _DS)],
            osems[bb]).wait()


@jax.jit
def _run(tbl, idx3):
    mesh = plsc.VectorSubcoreMesh(core_axis_name="c", subcore_axis_name="s")
    f = functools.partial(
        pl.kernel,
        out_type=jax.ShapeDtypeStruct((_B, _L, _D), jnp.float32),
        mesh=mesh,
        scratch_types=[
            pltpu.VMEM((_V, _DS), jnp.float32),
            pltpu.VMEM((_IGRP * _L * 3 + 16,), jnp.int32),
            pltpu.VMEM((_L, _DS), jnp.float32),
            pltpu.VMEM((_L, _DS), jnp.float32),
            pltpu.SemaphoreType.DMA,
            pltpu.SemaphoreType.DMA,
        ],
    )(_sc_body)
    return f(tbl, idx3)


def kernel(val_table, coord_table, pos_table, input_ids, coord_ids, pos_ids):
    tbl = jnp.concatenate(
        [val_table, coord_table, pos_table,
         jnp.zeros((_V - 219, _D), jnp.float32)], axis=0)
    iv = input_ids.reshape(-1).astype(jnp.int32)
    ic = coord_ids.reshape(-1).astype(jnp.int32) + _NVAL
    ip = pos_ids.reshape(-1).astype(jnp.int32) + (_NVAL + _NCOORD)
    idx3 = jnp.stack([iv, ic, ip], axis=1).reshape(-1)
    return _run(tbl, idx3)
